# baseline (speedup 1.0000x reference)
import jax
import jax.numpy as jnp
from jax import lax
from jax.experimental import pallas as pl


_RB = 32
_CHUNK = 256
_BIG = 2**30


def _body(x_ref, o_ref, idx_ref):
    n = x_ref.shape[1]
    nchunks = n // _CHUNK

    o_ref[...] = x_ref[...]

    def step(j, carry):
        runmin, runj = carry
        v = x_ref[:, pl.ds(j * _CHUNK, _CHUNK)]
        lt = v < runmin
        runj = jnp.where(lt, j, runj)
        runmin = jnp.minimum(runmin, v)
        return runmin, runj

    init = (
        jnp.full((_RB, _CHUNK), jnp.inf, jnp.float32),
        jnp.zeros((_RB, _CHUNK), jnp.int32),
    )
    runmin, runj = lax.fori_loop(0, nchunks, step, init, unroll=4)

    m = jnp.min(runmin, axis=1, keepdims=True)
    slot = lax.broadcasted_iota(jnp.int32, (_RB, _CHUNK), 1)
    cand = jnp.where(runmin == m, runj * _CHUNK + slot, _BIG)
    idx_ref[...] = jnp.min(cand, axis=1, keepdims=True)


def kernel(inputs):
    m, n = inputs.shape
    out, idx = pl.pallas_call(
        _body,
        grid=(m // _RB,),
        in_specs=[pl.BlockSpec((_RB, n), lambda i: (i, 0))],
        out_specs=[
            pl.BlockSpec((_RB, n), lambda i: (i, 0)),
            pl.BlockSpec((_RB, 1), lambda i: (i, 0)),
        ],
        out_shape=[
            jax.ShapeDtypeStruct((m, n), inputs.dtype),
            jax.ShapeDtypeStruct((m, 1), jnp.int32),
        ],
    )(inputs)
    del idx
    return out


# speedup vs baseline: 1.0213x; 1.0213x over previous
import jax
import jax.numpy as jnp
from jax import lax
from jax.experimental import pallas as pl


_RB = 32
_CHUNK = 512
_BIG = 2**30


def _body(x_ref, o_ref, idx_ref):
    n = x_ref.shape[1]
    nchunks = n // _CHUNK

    def step(j, carry):
        runmin, runj = carry
        v = x_ref[:, pl.ds(j * _CHUNK, _CHUNK)]
        o_ref[:, pl.ds(j * _CHUNK, _CHUNK)] = v
        lt = v < runmin
        runj = jnp.where(lt, j, runj)
        runmin = jnp.minimum(runmin, v)
        return runmin, runj

    init = (
        jnp.full((_RB, _CHUNK), jnp.inf, jnp.float32),
        jnp.zeros((_RB, _CHUNK), jnp.int32),
    )
    runmin, runj = lax.fori_loop(0, nchunks, step, init, unroll=8)

    m = jnp.min(runmin, axis=1, keepdims=True)
    slot = lax.broadcasted_iota(jnp.int32, (_RB, _CHUNK), 1)
    cand = jnp.where(runmin == m, runj * _CHUNK + slot, _BIG)
    idx_ref[...] = jnp.min(cand, axis=1, keepdims=True)


def kernel(inputs):
    m, n = inputs.shape
    out, idx = pl.pallas_call(
        _body,
        grid=(m // _RB,),
        in_specs=[pl.BlockSpec((_RB, n), lambda i: (i, 0))],
        out_specs=[
            pl.BlockSpec((_RB, n), lambda i: (i, 0)),
            pl.BlockSpec((_RB, 1), lambda i: (i, 0)),
        ],
        out_shape=[
            jax.ShapeDtypeStruct((m, n), inputs.dtype),
            jax.ShapeDtypeStruct((m, 1), jnp.int32),
        ],
    )(inputs)
    del idx
    return out


# speedup vs baseline: 1.0280x; 1.0066x over previous
import jax
import jax.numpy as jnp
from jax import lax
from jax.experimental import pallas as pl


_RB = 32
_CHUNK = 512
_BIG = 2**30


def _body(x_ref, o_ref, idx_ref):
    n = x_ref.shape[1]
    nchunks = n // _CHUNK

    def step(j, carry):
        runmin, runj = carry
        v = x_ref[:, pl.ds(j * _CHUNK, _CHUNK)]
        o_ref[:, pl.ds(j * _CHUNK, _CHUNK)] = v
        lt = v < runmin
        runj = jnp.where(lt, j, runj)
        runmin = jnp.minimum(runmin, v)
        return runmin, runj

    init = (
        jnp.full((_RB, _CHUNK), jnp.inf, jnp.float32),
        jnp.zeros((_RB, _CHUNK), jnp.int32),
    )
    runmin, runj = lax.fori_loop(0, nchunks, step, init, unroll=4)

    m = jnp.min(runmin, axis=1, keepdims=True)
    slot = lax.broadcasted_iota(jnp.int32, (_RB, _CHUNK), 1)
    cand = jnp.where(runmin == m, runj * _CHUNK + slot, _BIG)
    idx_ref[...] = jnp.min(cand, axis=1, keepdims=True)


def kernel(inputs):
    m, n = inputs.shape
    out, idx = pl.pallas_call(
        _body,
        grid=(m // _RB,),
        in_specs=[pl.BlockSpec((_RB, n), lambda i: (i, 0))],
        out_specs=[
            pl.BlockSpec((_RB, n), lambda i: (i, 0)),
            pl.BlockSpec((_RB, 1), lambda i: (i, 0)),
        ],
        out_shape=[
            jax.ShapeDtypeStruct((m, n), inputs.dtype),
            jax.ShapeDtypeStruct((m, 1), jnp.int32),
        ],
    )(inputs)
    del idx
    return out
